# baseline (speedup 1.0000x reference)
import functools

import jax
import jax.numpy as jnp
from jax import lax
from jax.experimental import pallas as pl
from jax.experimental.pallas import tpu as pltpu, tpu_sc as plsc

B, S, D = 1, 2048, 768
E, F, K = 64, 384, 8
T = B * S
TM = 128
NT = T * K // TM + E
NTP = 256
NS = NT * TM
CH = 256
A = T * K
NW = 32
APW = A // NW
GC = 64
NCH = APW // GC
HD = D // 2
DW = HD

_mesh = plsc.VectorSubcoreMesh(core_axis_name="c", subcore_axis_name="s")


def _wid():
    return lax.axis_index("s") * 2 + lax.axis_index("c")


def _pack16(lo_bf16, hi_bf16):
    lo = jax.lax.bitcast_convert_type(lo_bf16, jnp.int16).astype(jnp.int32)
    hi = jax.lax.bitcast_convert_type(hi_bf16, jnp.int16).astype(jnp.int32)
    return (lo & 0xFFFF) | (hi << 16)


def _unpack16(v):
    lo_f = jax.lax.bitcast_convert_type(v << 16, jnp.float32)
    hi_f = jax.lax.bitcast_convert_type(v & jnp.int32(-65536), jnp.float32)
    return lo_f, hi_f


def _router_meta_kernel(x_ref, wg_ref, scores_ref, pos_ref, gid_ref, nt_ref,
                        xi_ref, m_ref, r_ref):
    xf = x_ref[...]
    xi_ref[...] = _pack16(xf[:, :HD].astype(jnp.bfloat16),
                          xf[:, HD:].astype(jnp.bfloat16))
    logits = jax.lax.dot_general(
        xf, wg_ref[...], (((1,), (1,)), ((), ())),
        preferred_element_type=jnp.float32)
    mx = jnp.max(logits, axis=-1, keepdims=True)
    ex = jnp.exp(logits - mx)
    probs = ex / jnp.sum(ex, axis=-1, keepdims=True)

    lane = jax.lax.broadcasted_iota(jnp.int32, (T, E), 1)
    cur = probs
    sel_any = jnp.zeros((T, E), dtype=jnp.float32)
    eidx_cols = []
    score_cols = []
    for _ in range(K):
        m = jnp.max(cur, axis=-1, keepdims=True)
        is_max = cur == m
        first = jnp.min(jnp.where(is_max, lane, E), axis=-1, keepdims=True)
        sel = lane == first
        eidx_cols.append(first)
        score_cols.append(
            jnp.sum(jnp.where(sel, probs, 0.0), axis=-1, keepdims=True))
        sel_any = jnp.where(sel, 1.0, sel_any)
        cur = jnp.where(sel, -jnp.inf, cur)
    eidx = jnp.concatenate(eidx_cols, axis=1)
    sc = jnp.concatenate(score_cols, axis=1)
    scores_ref[...] = sc / jnp.sum(sc, axis=1, keepdims=True)

    m_ref[...] = sel_any
    row = jax.lax.broadcasted_iota(jnp.int32, (CH, CH), 0)
    col = jax.lax.broadcasted_iota(jnp.int32, (CH, CH), 1)
    tril_s = jnp.where(col < row, 1.0, 0.0)

    def body(k, base):
        mc = m_ref[pl.ds(k * CH, CH), :]
        rc = jax.lax.dot_general(
            tril_s, mc, (((1,), (0,)), ((), ())),
            preferred_element_type=jnp.float32) + base
        r_ref[pl.ds(k * CH, CH), :] = rc
        return base + jnp.sum(mc, axis=0, keepdims=True)

    counts = jax.lax.fori_loop(0, T // CH, body, jnp.zeros((1, E), jnp.float32))

    tcnt = jnp.ceil(counts / TM)
    er = jax.lax.broadcasted_iota(jnp.int32, (E, E), 0)
    ec = jax.lax.broadcasted_iota(jnp.int32, (E, E), 1)
    upper_s = jnp.where(er < ec, 1.0, 0.0)
    ts_row = jax.lax.dot_general(
        tcnt, upper_s, (((1,), (0,)), ((), ())),
        preferred_element_type=jnp.float32)
    nt_ref[...] = jnp.sum(tcnt, axis=1, keepdims=True).astype(jnp.int32)

    lower_s = jnp.where(er > ec, 1.0, 0.0)
    ts_col = jax.lax.dot_general(
        lower_s, tcnt, (((1,), (1,)), ((), ())),
        preferred_element_type=jnp.float32)
    ti = jax.lax.broadcasted_iota(jnp.int32, (E, NTP), 1)
    cmp = jnp.where(ti >= ts_col.astype(jnp.int32), 1.0, 0.0)
    gid = jax.lax.dot_general(
        jnp.ones((1, E), jnp.float32), cmp, (((1,), (0,)), ((), ())),
        preferred_element_type=jnp.float32) - 1.0
    gid_ref[...] = jnp.clip(gid, 0.0, E - 1).astype(jnp.int32)

    rmat = r_ref[...]
    base_row = ts_row * TM
    for k in range(K):
        sel = lane == eidx[:, k:k + 1]
        posk = jnp.sum(jnp.where(sel, base_row + rmat, 0.0),
                       axis=1, keepdims=True)
        pos_ref[:, k:k + 1] = posk.astype(jnp.int32)


def _router_meta(x, Wg):
    return pl.pallas_call(
        _router_meta_kernel,
        out_shape=(
            jax.ShapeDtypeStruct((T, K), jnp.float32),
            jax.ShapeDtypeStruct((T, K), jnp.int32),
            jax.ShapeDtypeStruct((1, NTP), jnp.int32),
            jax.ShapeDtypeStruct((1, 1), jnp.int32),
            jax.ShapeDtypeStruct((T, DW), jnp.int32),
        ),
        scratch_shapes=[
            pltpu.VMEM((T, E), jnp.float32),
            pltpu.VMEM((T, E), jnp.float32),
        ],
    )(x, Wg)


@functools.partial(
    pl.kernel, mesh=_mesh,
    out_type=jax.ShapeDtypeStruct((NS, DW), jnp.int32),
    scratch_types=[
        pltpu.VMEM((NCH, GC), jnp.int32),
        pltpu.VMEM((APW,), jnp.int32),
        pltpu.VMEM((2, GC, DW), jnp.int32),
        pltpu.SemaphoreType.DMA,
        pltpu.SemaphoreType.DMA,
    ],
)
def _sc_dispatch(pos_hbm, x_hbm, xs_hbm, pos_v, tik_v, buf, gsem, ssem):
    base = _wid() * APW
    for c in range(NCH):
        pltpu.sync_copy(pos_hbm.at[pl.ds(base + c * GC, GC)], pos_v.at[c])
    for q in range(APW // 16):
        v = lax.broadcasted_iota(jnp.int32, (16,), 0)
        tik_v[pl.ds(q * 16, 16)] = (v + (base + q * 16)) >> 3

    def gather(c):
        return pltpu.async_copy(
            x_hbm.at[tik_v.at[pl.ds(c * GC, GC)]], buf.at[c % 2], gsem)

    def scatter(c):
        return pltpu.async_copy(
            buf.at[c % 2], xs_hbm.at[pos_v.at[c]], ssem)

    gather(0)
    for c in range(NCH):
        pltpu.make_async_copy(
            x_hbm.at[tik_v.at[pl.ds(c * GC, GC)]], buf.at[c % 2], gsem).wait()
        scatter(c)
        if c + 1 < NCH:
            if c >= 1:
                pltpu.make_async_copy(
                    buf.at[(c - 1) % 2], xs_hbm.at[pos_v.at[c - 1]],
                    ssem).wait()
            gather(c + 1)
    for c in range(NCH - 2, NCH):
        pltpu.make_async_copy(
            buf.at[c % 2], xs_hbm.at[pos_v.at[c]], ssem).wait()


def _ffn_kernel(gid_ref, nt_ref, xs_ref, w1_ref, w3_ref, w2_ref, ys_ref):
    i = pl.program_id(0)

    @pl.when(i < nt_ref[0])
    def _():
        lo_f, hi_f = _unpack16(xs_ref[...])
        xlo = lo_f.astype(jnp.bfloat16)
        xhi = hi_f.astype(jnp.bfloat16)

        def dot2(w_ref):
            return (jax.lax.dot_general(
                        xlo, w_ref[0, :HD, :], (((1,), (0,)), ((), ())),
                        preferred_element_type=jnp.float32)
                    + jax.lax.dot_general(
                        xhi, w_ref[0, HD:, :], (((1,), (0,)), ((), ())),
                        preferred_element_type=jnp.float32))

        h1 = dot2(w1_ref)
        h3 = dot2(w3_ref)
        h = (h1 * jax.lax.logistic(h1) * h3).astype(jnp.bfloat16)
        y = jax.lax.dot_general(
            h, w2_ref[0], (((1,), (0,)), ((), ())),
            preferred_element_type=jnp.float32)
        ys_ref[...] = _pack16(y[:, :HD].astype(jnp.bfloat16),
                              y[:, HD:].astype(jnp.bfloat16))


def _grouped_ffn(xs, W1b, W3b, W2b, gid, nt):
    grid_spec = pltpu.PrefetchScalarGridSpec(
        num_scalar_prefetch=2,
        grid=(NT,),
        in_specs=[
            pl.BlockSpec((TM, DW), lambda i, g, n: (i, 0)),
            pl.BlockSpec((1, D, F), lambda i, g, n: (g[i], 0, 0)),
            pl.BlockSpec((1, D, F), lambda i, g, n: (g[i], 0, 0)),
            pl.BlockSpec((1, F, D), lambda i, g, n: (g[i], 0, 0)),
        ],
        out_specs=pl.BlockSpec((TM, DW), lambda i, g, n: (i, 0)),
    )
    return pl.pallas_call(
        _ffn_kernel,
        grid_spec=grid_spec,
        out_shape=jax.ShapeDtypeStruct((NS, DW), jnp.int32),
        compiler_params=pltpu.CompilerParams(
            dimension_semantics=("arbitrary",),
        ),
    )(gid, nt, xs, W1b, W3b, W2b)


@functools.partial(
    pl.kernel, mesh=_mesh,
    out_type=jax.ShapeDtypeStruct((A, DW), jnp.int32),
    scratch_types=[
        pltpu.VMEM((APW,), jnp.int32),
        pltpu.VMEM((2, GC, DW), jnp.int32),
        pltpu.SemaphoreType.DMA,
        pltpu.SemaphoreType.DMA,
    ],
)
def _sc_gather_back(pos_hbm, ys_hbm, yg_hbm, pos_v, buf, gsem, wsem):
    base = _wid() * APW
    pltpu.sync_copy(pos_hbm.at[pl.ds(base, APW)], pos_v)

    def gather(c):
        return pltpu.async_copy(
            ys_hbm.at[pos_v.at[pl.ds(c * GC, GC)]], buf.at[c % 2], gsem)

    def wout(c):
        return pltpu.async_copy(
            buf.at[c % 2], yg_hbm.at[pl.ds(base + c * GC, GC)], wsem)

    gather(0)
    for c in range(NCH):
        pltpu.make_async_copy(
            ys_hbm.at[pos_v.at[pl.ds(c * GC, GC)]], buf.at[c % 2], gsem).wait()
        wout(c)
        if c + 1 < NCH:
            if c >= 1:
                pltpu.make_async_copy(
                    buf.at[(c - 1) % 2],
                    yg_hbm.at[pl.ds(base + (c - 1) * GC, GC)], wsem).wait()
            gather(c + 1)
    for c in range(NCH - 2, NCH):
        pltpu.make_async_copy(
            buf.at[c % 2], yg_hbm.at[pl.ds(base + c * GC, GC)], wsem).wait()


def _combine_kernel(yg_ref, sc_ref, out_ref):
    lo_f, hi_f = _unpack16(yg_ref[:, 0, :])
    acc_lo = sc_ref[:, 0:1] * lo_f
    acc_hi = sc_ref[:, 0:1] * hi_f
    for k in range(1, K):
        lo_f, hi_f = _unpack16(yg_ref[:, k, :])
        acc_lo = acc_lo + sc_ref[:, k:k + 1] * lo_f
        acc_hi = acc_hi + sc_ref[:, k:k + 1] * hi_f
    out_ref[:, :HD] = acc_lo
    out_ref[:, HD:] = acc_hi


def _combine(yg, scores):
    CB = 256
    return pl.pallas_call(
        _combine_kernel,
        grid=(T // CB,),
        in_specs=[
            pl.BlockSpec((CB, K, DW), lambda i: (i, 0, 0)),
            pl.BlockSpec((CB, K), lambda i: (i, 0)),
        ],
        out_specs=pl.BlockSpec((CB, D), lambda i: (i, 0)),
        out_shape=jax.ShapeDtypeStruct((T, D), jnp.float32),
    )(yg, scores)


def kernel(hidden_states, Wg, W1, W3, W2):
    old_shape = hidden_states.shape
    x = hidden_states.reshape(-1, old_shape[-1])

    scores, pos, gid, nt, xi = _router_meta(x, Wg)
    gid = gid.reshape(NTP)
    nt = nt.reshape(1)
    pos_flat = pos.reshape(A)

    xs = _sc_dispatch(pos_flat, xi)
    ys = _grouped_ffn(xs, W1.astype(jnp.bfloat16), W3.astype(jnp.bfloat16),
                      W2.astype(jnp.bfloat16), gid, nt)
    yg = _sc_gather_back(pos_flat, ys)
    out = _combine(yg.reshape(T, K, DW), scores)
    return out.reshape(old_shape)


# speedup vs baseline: 1.2122x; 1.2122x over previous
import functools

import jax
import jax.numpy as jnp
from jax import lax
from jax.experimental import pallas as pl
from jax.experimental.pallas import tpu as pltpu, tpu_sc as plsc

B, S, D = 1, 2048, 768
E, F, K = 64, 384, 8
T = B * S
TM = 128
NT = T * K // TM + E
NTP = 256
NS = NT * TM
CH = 256
A = T * K
NW = 32
APW = A // NW
GC = 64
NCH = APW // GC
HD = D // 2
DW = HD

_mesh = plsc.VectorSubcoreMesh(core_axis_name="c", subcore_axis_name="s")


def _wid():
    return lax.axis_index("s") * 2 + lax.axis_index("c")


def _pack16(lo_bf16, hi_bf16):
    lo = jax.lax.bitcast_convert_type(lo_bf16, jnp.int16).astype(jnp.int32)
    hi = jax.lax.bitcast_convert_type(hi_bf16, jnp.int16).astype(jnp.int32)
    return (lo & 0xFFFF) | (hi << 16)


def _unpack16(v):
    lo_f = jax.lax.bitcast_convert_type(v << 16, jnp.float32)
    hi_f = jax.lax.bitcast_convert_type(v & jnp.int32(-65536), jnp.float32)
    return lo_f, hi_f


def _router_meta_kernel(x_ref, wg_ref, scores_ref, pos_ref, gid_ref, nt_ref,
                        xi_ref, m_ref, r_ref):
    xf = x_ref[...]
    xi_ref[...] = _pack16(xf[:, :HD].astype(jnp.bfloat16),
                          xf[:, HD:].astype(jnp.bfloat16))
    logits = jax.lax.dot_general(
        xf, wg_ref[...], (((1,), (1,)), ((), ())),
        preferred_element_type=jnp.float32)
    mx = jnp.max(logits, axis=-1, keepdims=True)
    ex = jnp.exp(logits - mx)
    probs = ex / jnp.sum(ex, axis=-1, keepdims=True)

    lane = jax.lax.broadcasted_iota(jnp.int32, (T, E), 1)
    cur = probs
    sel_any = jnp.zeros((T, E), dtype=jnp.float32)
    eidx_cols = []
    score_cols = []
    for _ in range(K):
        m = jnp.max(cur, axis=-1, keepdims=True)
        is_max = cur == m
        first = jnp.min(jnp.where(is_max, lane, E), axis=-1, keepdims=True)
        sel = lane == first
        eidx_cols.append(first)
        score_cols.append(
            jnp.sum(jnp.where(sel, probs, 0.0), axis=-1, keepdims=True))
        sel_any = jnp.where(sel, 1.0, sel_any)
        cur = jnp.where(sel, -jnp.inf, cur)
    eidx = jnp.concatenate(eidx_cols, axis=1)
    sc = jnp.concatenate(score_cols, axis=1)
    scores_ref[...] = sc / jnp.sum(sc, axis=1, keepdims=True)

    m_ref[...] = sel_any
    row = jax.lax.broadcasted_iota(jnp.int32, (CH, CH), 0)
    col = jax.lax.broadcasted_iota(jnp.int32, (CH, CH), 1)
    tril_s = jnp.where(col < row, 1.0, 0.0)

    def body(k, base):
        mc = m_ref[pl.ds(k * CH, CH), :]
        rc = jax.lax.dot_general(
            tril_s, mc, (((1,), (0,)), ((), ())),
            preferred_element_type=jnp.float32) + base
        r_ref[pl.ds(k * CH, CH), :] = rc
        return base + jnp.sum(mc, axis=0, keepdims=True)

    counts = jax.lax.fori_loop(0, T // CH, body, jnp.zeros((1, E), jnp.float32))

    tcnt = jnp.ceil(counts / TM)
    er = jax.lax.broadcasted_iota(jnp.int32, (E, E), 0)
    ec = jax.lax.broadcasted_iota(jnp.int32, (E, E), 1)
    upper_s = jnp.where(er < ec, 1.0, 0.0)
    ts_row = jax.lax.dot_general(
        tcnt, upper_s, (((1,), (0,)), ((), ())),
        preferred_element_type=jnp.float32)
    nt_ref[...] = jnp.sum(tcnt, axis=1, keepdims=True).astype(jnp.int32)

    lower_s = jnp.where(er > ec, 1.0, 0.0)
    ts_col = jax.lax.dot_general(
        lower_s, tcnt, (((1,), (1,)), ((), ())),
        preferred_element_type=jnp.float32)
    ti = jax.lax.broadcasted_iota(jnp.int32, (E, NTP), 1)
    cmp = jnp.where(ti >= ts_col.astype(jnp.int32), 1.0, 0.0)
    gid = jax.lax.dot_general(
        jnp.ones((1, E), jnp.float32), cmp, (((1,), (0,)), ((), ())),
        preferred_element_type=jnp.float32) - 1.0
    gid_ref[...] = jnp.clip(gid, 0.0, E - 1).astype(jnp.int32)

    rmat = r_ref[...]
    base_row = ts_row * TM
    for k in range(K):
        sel = lane == eidx[:, k:k + 1]
        posk = jnp.sum(jnp.where(sel, base_row + rmat, 0.0),
                       axis=1, keepdims=True)
        pos_ref[:, k:k + 1] = posk.astype(jnp.int32)


def _router_meta(x, Wg):
    return pl.pallas_call(
        _router_meta_kernel,
        out_shape=(
            jax.ShapeDtypeStruct((T, K), jnp.float32),
            jax.ShapeDtypeStruct((T, K), jnp.int32),
            jax.ShapeDtypeStruct((1, NTP), jnp.int32),
            jax.ShapeDtypeStruct((1, 1), jnp.int32),
            jax.ShapeDtypeStruct((T, DW), jnp.int32),
        ),
        scratch_shapes=[
            pltpu.VMEM((T, E), jnp.float32),
            pltpu.VMEM((T, E), jnp.float32),
        ],
    )(x, Wg)


@functools.partial(
    pl.kernel, mesh=_mesh,
    out_type=jax.ShapeDtypeStruct((NS, DW), jnp.int32),
    scratch_types=[
        pltpu.VMEM((NCH, GC), jnp.int32),
        pltpu.VMEM((APW,), jnp.int32),
        pltpu.VMEM((2, GC, DW), jnp.int32),
        pltpu.SemaphoreType.DMA,
        pltpu.SemaphoreType.DMA,
    ],
)
def _sc_dispatch(pos_hbm, x_hbm, xs_hbm, pos_v, tik_v, buf, gsem, ssem):
    base = _wid() * APW
    for c in range(NCH):
        pltpu.sync_copy(pos_hbm.at[pl.ds(base + c * GC, GC)], pos_v.at[c])
    for q in range(APW // 16):
        v = lax.broadcasted_iota(jnp.int32, (16,), 0)
        tik_v[pl.ds(q * 16, 16)] = (v + (base + q * 16)) >> 3

    def gather(c):
        return pltpu.async_copy(
            x_hbm.at[tik_v.at[pl.ds(c * GC, GC)]], buf.at[c % 2], gsem)

    def scatter(c):
        return pltpu.async_copy(
            buf.at[c % 2], xs_hbm.at[pos_v.at[c]], ssem)

    gather(0)
    for c in range(NCH):
        pltpu.make_async_copy(
            x_hbm.at[tik_v.at[pl.ds(c * GC, GC)]], buf.at[c % 2], gsem).wait()
        scatter(c)
        if c + 1 < NCH:
            if c >= 1:
                pltpu.make_async_copy(
                    buf.at[(c - 1) % 2], xs_hbm.at[pos_v.at[c - 1]],
                    ssem).wait()
            gather(c + 1)
    for c in range(NCH - 2, NCH):
        pltpu.make_async_copy(
            buf.at[c % 2], xs_hbm.at[pos_v.at[c]], ssem).wait()


def _ffn_kernel(gid_ref, nt_ref, xs_ref, w1_ref, w3_ref, w2_ref, ys_ref):
    i = pl.program_id(0)

    @pl.when(i < nt_ref[0])
    def _():
        lo_f, hi_f = _unpack16(xs_ref[...])
        xlo = lo_f.astype(jnp.bfloat16)
        xhi = hi_f.astype(jnp.bfloat16)

        def dot2(w_ref):
            return (jax.lax.dot_general(
                        xlo, w_ref[0, :HD, :].astype(jnp.bfloat16),
                        (((1,), (0,)), ((), ())),
                        preferred_element_type=jnp.float32)
                    + jax.lax.dot_general(
                        xhi, w_ref[0, HD:, :].astype(jnp.bfloat16),
                        (((1,), (0,)), ((), ())),
                        preferred_element_type=jnp.float32))

        h1 = dot2(w1_ref)
        h3 = dot2(w3_ref)
        h = (h1 * jax.lax.logistic(h1) * h3).astype(jnp.bfloat16)
        y = jax.lax.dot_general(
            h, w2_ref[0].astype(jnp.bfloat16), (((1,), (0,)), ((), ())),
            preferred_element_type=jnp.float32)
        ys_ref[...] = _pack16(y[:, :HD].astype(jnp.bfloat16),
                              y[:, HD:].astype(jnp.bfloat16))


def _grouped_ffn(xs, W1b, W3b, W2b, gid, nt):
    grid_spec = pltpu.PrefetchScalarGridSpec(
        num_scalar_prefetch=2,
        grid=(NT,),
        in_specs=[
            pl.BlockSpec((TM, DW), lambda i, g, n: (i, 0)),
            pl.BlockSpec((1, D, F), lambda i, g, n: (g[i], 0, 0)),
            pl.BlockSpec((1, D, F), lambda i, g, n: (g[i], 0, 0)),
            pl.BlockSpec((1, F, D), lambda i, g, n: (g[i], 0, 0)),
        ],
        out_specs=pl.BlockSpec((TM, DW), lambda i, g, n: (i, 0)),
    )
    return pl.pallas_call(
        _ffn_kernel,
        grid_spec=grid_spec,
        out_shape=jax.ShapeDtypeStruct((NS, DW), jnp.int32),
        compiler_params=pltpu.CompilerParams(
            dimension_semantics=("arbitrary",),
        ),
    )(gid, nt, xs, W1b, W3b, W2b)


@functools.partial(
    pl.kernel, mesh=_mesh,
    out_type=jax.ShapeDtypeStruct((A, DW), jnp.int32),
    scratch_types=[
        pltpu.VMEM((APW,), jnp.int32),
        pltpu.VMEM((2, GC, DW), jnp.int32),
        pltpu.SemaphoreType.DMA,
        pltpu.SemaphoreType.DMA,
    ],
)
def _sc_gather_back(pos_hbm, ys_hbm, yg_hbm, pos_v, buf, gsem, wsem):
    base = _wid() * APW
    pltpu.sync_copy(pos_hbm.at[pl.ds(base, APW)], pos_v)

    def gather(c):
        return pltpu.async_copy(
            ys_hbm.at[pos_v.at[pl.ds(c * GC, GC)]], buf.at[c % 2], gsem)

    def wout(c):
        return pltpu.async_copy(
            buf.at[c % 2], yg_hbm.at[pl.ds(base + c * GC, GC)], wsem)

    gather(0)
    for c in range(NCH):
        pltpu.make_async_copy(
            ys_hbm.at[pos_v.at[pl.ds(c * GC, GC)]], buf.at[c % 2], gsem).wait()
        wout(c)
        if c + 1 < NCH:
            if c >= 1:
                pltpu.make_async_copy(
                    buf.at[(c - 1) % 2],
                    yg_hbm.at[pl.ds(base + (c - 1) * GC, GC)], wsem).wait()
            gather(c + 1)
    for c in range(NCH - 2, NCH):
        pltpu.make_async_copy(
            buf.at[c % 2], yg_hbm.at[pl.ds(base + c * GC, GC)], wsem).wait()


def _combine_kernel(yg_ref, sc_ref, out_ref):
    lo_f, hi_f = _unpack16(yg_ref[:, 0:DW])
    acc_lo = sc_ref[:, 0:1] * lo_f
    acc_hi = sc_ref[:, 0:1] * hi_f
    for k in range(1, K):
        lo_f, hi_f = _unpack16(yg_ref[:, k * DW:(k + 1) * DW])
        acc_lo = acc_lo + sc_ref[:, k:k + 1] * lo_f
        acc_hi = acc_hi + sc_ref[:, k:k + 1] * hi_f
    out_ref[:, :HD] = acc_lo
    out_ref[:, HD:] = acc_hi


def _combine(yg, scores):
    CB = 256
    return pl.pallas_call(
        _combine_kernel,
        grid=(T // CB,),
        in_specs=[
            pl.BlockSpec((CB, K * DW), lambda i: (i, 0)),
            pl.BlockSpec((CB, K), lambda i: (i, 0)),
        ],
        out_specs=pl.BlockSpec((CB, D), lambda i: (i, 0)),
        out_shape=jax.ShapeDtypeStruct((T, D), jnp.float32),
    )(yg, scores)


def kernel(hidden_states, Wg, W1, W3, W2):
    old_shape = hidden_states.shape
    x = hidden_states.reshape(-1, old_shape[-1])

    scores, pos, gid, nt, xi = _router_meta(x, Wg)
    gid = gid.reshape(NTP)
    nt = nt.reshape(1)
    pos_flat = pos.reshape(A)

    xs = _sc_dispatch(pos_flat, xi)
    ys = _grouped_ffn(xs, W1, W3, W2, gid, nt)
    yg = _sc_gather_back(pos_flat, ys)
    out = _combine(yg.reshape(T, K * DW), scores)
    return out.reshape(old_shape)


# speedup vs baseline: 1.4185x; 1.1702x over previous
import functools

import jax
import jax.numpy as jnp
from jax import lax
from jax.experimental import pallas as pl
from jax.experimental.pallas import tpu as pltpu, tpu_sc as plsc

B, S, D = 1, 2048, 768
E, F, K = 64, 384, 8
T = B * S
TM = 256
NT = T * K // TM + E
NTP = 256
NS = NT * TM
CH = 256
A = T * K
NW = 32
APW = A // NW
GC = 64
NCH = APW // GC
HD = D // 2
DW = HD

_mesh = plsc.VectorSubcoreMesh(core_axis_name="c", subcore_axis_name="s")


def _wid():
    return lax.axis_index("s") * 2 + lax.axis_index("c")


def _pack16(lo_bf16, hi_bf16):
    lo = jax.lax.bitcast_convert_type(lo_bf16, jnp.int16).astype(jnp.int32)
    hi = jax.lax.bitcast_convert_type(hi_bf16, jnp.int16).astype(jnp.int32)
    return (lo & 0xFFFF) | (hi << 16)


def _unpack16(v):
    lo_f = jax.lax.bitcast_convert_type(v << 16, jnp.float32)
    hi_f = jax.lax.bitcast_convert_type(v & jnp.int32(-65536), jnp.float32)
    return lo_f, hi_f


def _router_meta_kernel(x_ref, wg_ref, scores_ref, pos_ref, gid_ref, nt_ref,
                        xi_ref, m_ref, r_ref):
    xf = x_ref[...]
    xi_ref[...] = _pack16(xf[:, :HD].astype(jnp.bfloat16),
                          xf[:, HD:].astype(jnp.bfloat16))
    logits = jax.lax.dot_general(
        xf, wg_ref[...], (((1,), (1,)), ((), ())),
        preferred_element_type=jnp.float32)
    mx = jnp.max(logits, axis=-1, keepdims=True)
    ex = jnp.exp(logits - mx)
    probs = ex / jnp.sum(ex, axis=-1, keepdims=True)

    lane = jax.lax.broadcasted_iota(jnp.int32, (T, E), 1)
    cur = probs
    sel_any = jnp.zeros((T, E), dtype=jnp.float32)
    eidx_cols = []
    score_cols = []
    for _ in range(K):
        m = jnp.max(cur, axis=-1, keepdims=True)
        is_max = cur == m
        first = jnp.min(jnp.where(is_max, lane, E), axis=-1, keepdims=True)
        sel = lane == first
        eidx_cols.append(first)
        score_cols.append(
            jnp.sum(jnp.where(sel, probs, 0.0), axis=-1, keepdims=True))
        sel_any = jnp.where(sel, 1.0, sel_any)
        cur = jnp.where(sel, -jnp.inf, cur)
    eidx = jnp.concatenate(eidx_cols, axis=1)
    sc = jnp.concatenate(score_cols, axis=1)
    scores_ref[...] = sc / jnp.sum(sc, axis=1, keepdims=True)

    m_ref[...] = sel_any
    row = jax.lax.broadcasted_iota(jnp.int32, (CH, CH), 0)
    col = jax.lax.broadcasted_iota(jnp.int32, (CH, CH), 1)
    tril_s = jnp.where(col < row, 1.0, 0.0)

    def body(k, base):
        mc = m_ref[pl.ds(k * CH, CH), :]
        rc = jax.lax.dot_general(
            tril_s, mc, (((1,), (0,)), ((), ())),
            preferred_element_type=jnp.float32) + base
        r_ref[pl.ds(k * CH, CH), :] = rc
        return base + jnp.sum(mc, axis=0, keepdims=True)

    counts = jax.lax.fori_loop(0, T // CH, body, jnp.zeros((1, E), jnp.float32))

    tcnt = jnp.ceil(counts / TM)
    er = jax.lax.broadcasted_iota(jnp.int32, (E, E), 0)
    ec = jax.lax.broadcasted_iota(jnp.int32, (E, E), 1)
    upper_s = jnp.where(er < ec, 1.0, 0.0)
    ts_row = jax.lax.dot_general(
        tcnt, upper_s, (((1,), (0,)), ((), ())),
        preferred_element_type=jnp.float32)
    nt_ref[...] = jnp.sum(tcnt, axis=1, keepdims=True).astype(jnp.int32)

    lower_s = jnp.where(er > ec, 1.0, 0.0)
    ts_col = jax.lax.dot_general(
        lower_s, tcnt, (((1,), (1,)), ((), ())),
        preferred_element_type=jnp.float32)
    ti = jax.lax.broadcasted_iota(jnp.int32, (E, NTP), 1)
    cmp = jnp.where(ti >= ts_col.astype(jnp.int32), 1.0, 0.0)
    gid = jax.lax.dot_general(
        jnp.ones((1, E), jnp.float32), cmp, (((1,), (0,)), ((), ())),
        preferred_element_type=jnp.float32) - 1.0
    gid_ref[...] = jnp.clip(gid, 0.0, E - 1).astype(jnp.int32)

    rmat = r_ref[...]
    base_row = ts_row * TM
    for k in range(K):
        sel = lane == eidx[:, k:k + 1]
        posk = jnp.sum(jnp.where(sel, base_row + rmat, 0.0),
                       axis=1, keepdims=True)
        pos_ref[:, k:k + 1] = posk.astype(jnp.int32)


def _router_meta(x, Wg):
    return pl.pallas_call(
        _router_meta_kernel,
        out_shape=(
            jax.ShapeDtypeStruct((T, K), jnp.float32),
            jax.ShapeDtypeStruct((T, K), jnp.int32),
            jax.ShapeDtypeStruct((1, NTP), jnp.int32),
            jax.ShapeDtypeStruct((1, 1), jnp.int32),
            jax.ShapeDtypeStruct((T, DW), jnp.int32),
        ),
        scratch_shapes=[
            pltpu.VMEM((T, E), jnp.float32),
            pltpu.VMEM((T, E), jnp.float32),
        ],
    )(x, Wg)


@functools.partial(
    pl.kernel, mesh=_mesh,
    out_type=jax.ShapeDtypeStruct((NS, DW), jnp.int32),
    scratch_types=[
        pltpu.VMEM((NCH, GC), jnp.int32),
        pltpu.VMEM((APW,), jnp.int32),
        pltpu.VMEM((2, GC, DW), jnp.int32),
        pltpu.SemaphoreType.DMA,
        pltpu.SemaphoreType.DMA,
    ],
)
def _sc_dispatch(pos_hbm, x_hbm, xs_hbm, pos_v, tik_v, buf, gsem, ssem):
    base = _wid() * APW
    for c in range(NCH):
        pltpu.sync_copy(pos_hbm.at[pl.ds(base + c * GC, GC)], pos_v.at[c])
    for q in range(APW // 16):
        v = lax.broadcasted_iota(jnp.int32, (16,), 0)
        tik_v[pl.ds(q * 16, 16)] = (v + (base + q * 16)) >> 3

    def gather(c):
        return pltpu.async_copy(
            x_hbm.at[tik_v.at[pl.ds(c * GC, GC)]], buf.at[c % 2], gsem)

    def scatter(c):
        return pltpu.async_copy(
            buf.at[c % 2], xs_hbm.at[pos_v.at[c]], ssem)

    gather(0)
    for c in range(NCH):
        pltpu.make_async_copy(
            x_hbm.at[tik_v.at[pl.ds(c * GC, GC)]], buf.at[c % 2], gsem).wait()
        scatter(c)
        if c + 1 < NCH:
            if c >= 1:
                pltpu.make_async_copy(
                    buf.at[(c - 1) % 2], xs_hbm.at[pos_v.at[c - 1]],
                    ssem).wait()
            gather(c + 1)
    for c in range(NCH - 2, NCH):
        pltpu.make_async_copy(
            buf.at[c % 2], xs_hbm.at[pos_v.at[c]], ssem).wait()


def _ffn_kernel(gid_ref, nt_ref, xs_ref, w1_ref, w3_ref, w2_ref, ys_ref):
    i = pl.program_id(0)

    @pl.when(i < nt_ref[0])
    def _():
        lo_f, hi_f = _unpack16(xs_ref[...])
        xlo = lo_f.astype(jnp.bfloat16)
        xhi = hi_f.astype(jnp.bfloat16)

        def dot2(w_ref):
            return (jax.lax.dot_general(
                        xlo, w_ref[0, :HD, :].astype(jnp.bfloat16),
                        (((1,), (0,)), ((), ())),
                        preferred_element_type=jnp.float32)
                    + jax.lax.dot_general(
                        xhi, w_ref[0, HD:, :].astype(jnp.bfloat16),
                        (((1,), (0,)), ((), ())),
                        preferred_element_type=jnp.float32))

        h1 = dot2(w1_ref)
        h3 = dot2(w3_ref)
        h = (h1 * jax.lax.logistic(h1) * h3).astype(jnp.bfloat16)
        y = jax.lax.dot_general(
            h, w2_ref[0].astype(jnp.bfloat16), (((1,), (0,)), ((), ())),
            preferred_element_type=jnp.float32)
        ys_ref[...] = _pack16(y[:, :HD].astype(jnp.bfloat16),
                              y[:, HD:].astype(jnp.bfloat16))


def _grouped_ffn(xs, W1b, W3b, W2b, gid, nt):
    grid_spec = pltpu.PrefetchScalarGridSpec(
        num_scalar_prefetch=2,
        grid=(NT,),
        in_specs=[
            pl.BlockSpec((TM, DW), lambda i, g, n: (i, 0)),
            pl.BlockSpec((1, D, F), lambda i, g, n: (g[i], 0, 0)),
            pl.BlockSpec((1, D, F), lambda i, g, n: (g[i], 0, 0)),
            pl.BlockSpec((1, F, D), lambda i, g, n: (g[i], 0, 0)),
        ],
        out_specs=pl.BlockSpec((TM, DW), lambda i, g, n: (i, 0)),
    )
    return pl.pallas_call(
        _ffn_kernel,
        grid_spec=grid_spec,
        out_shape=jax.ShapeDtypeStruct((NS, DW), jnp.int32),
        compiler_params=pltpu.CompilerParams(
            dimension_semantics=("arbitrary",),
        ),
    )(gid, nt, xs, W1b, W3b, W2b)


@functools.partial(
    pl.kernel, mesh=_mesh,
    out_type=jax.ShapeDtypeStruct((A, DW), jnp.int32),
    scratch_types=[
        pltpu.VMEM((APW,), jnp.int32),
        pltpu.VMEM((2, GC, DW), jnp.int32),
        pltpu.SemaphoreType.DMA,
        pltpu.SemaphoreType.DMA,
    ],
)
def _sc_gather_back(pos_hbm, ys_hbm, yg_hbm, pos_v, buf, gsem, wsem):
    base = _wid() * APW
    pltpu.sync_copy(pos_hbm.at[pl.ds(base, APW)], pos_v)

    def gather(c):
        return pltpu.async_copy(
            ys_hbm.at[pos_v.at[pl.ds(c * GC, GC)]], buf.at[c % 2], gsem)

    def wout(c):
        return pltpu.async_copy(
            buf.at[c % 2], yg_hbm.at[pl.ds(base + c * GC, GC)], wsem)

    gather(0)
    for c in range(NCH):
        pltpu.make_async_copy(
            ys_hbm.at[pos_v.at[pl.ds(c * GC, GC)]], buf.at[c % 2], gsem).wait()
        wout(c)
        if c + 1 < NCH:
            if c >= 1:
                pltpu.make_async_copy(
                    buf.at[(c - 1) % 2],
                    yg_hbm.at[pl.ds(base + (c - 1) * GC, GC)], wsem).wait()
            gather(c + 1)
    for c in range(NCH - 2, NCH):
        pltpu.make_async_copy(
            buf.at[c % 2], yg_hbm.at[pl.ds(base + c * GC, GC)], wsem).wait()


def _combine_kernel(yg_ref, sc_ref, out_ref):
    lo_f, hi_f = _unpack16(yg_ref[:, 0:DW])
    acc_lo = sc_ref[:, 0:1] * lo_f
    acc_hi = sc_ref[:, 0:1] * hi_f
    for k in range(1, K):
        lo_f, hi_f = _unpack16(yg_ref[:, k * DW:(k + 1) * DW])
        acc_lo = acc_lo + sc_ref[:, k:k + 1] * lo_f
        acc_hi = acc_hi + sc_ref[:, k:k + 1] * hi_f
    out_ref[:, :HD] = acc_lo
    out_ref[:, HD:] = acc_hi


def _combine(yg, scores):
    CB = 256
    return pl.pallas_call(
        _combine_kernel,
        grid=(T // CB,),
        in_specs=[
            pl.BlockSpec((CB, K * DW), lambda i: (i, 0)),
            pl.BlockSpec((CB, K), lambda i: (i, 0)),
        ],
        out_specs=pl.BlockSpec((CB, D), lambda i: (i, 0)),
        out_shape=jax.ShapeDtypeStruct((T, D), jnp.float32),
    )(yg, scores)


def kernel(hidden_states, Wg, W1, W3, W2):
    old_shape = hidden_states.shape
    x = hidden_states.reshape(-1, old_shape[-1])

    scores, pos, gid, nt, xi = _router_meta(x, Wg)
    gid = gid.reshape(NTP)
    nt = nt.reshape(1)
    pos_flat = pos.reshape(A)

    xs = _sc_dispatch(pos_flat, xi)
    ys = _grouped_ffn(xs, W1, W3, W2, gid, nt)
    yg = _sc_gather_back(pos_flat, ys)
    out = _combine(yg.reshape(T, K * DW), scores)
    return out.reshape(old_shape)


# speedup vs baseline: 1.4418x; 1.0164x over previous
import functools

import jax
import jax.numpy as jnp
from jax import lax
from jax.experimental import pallas as pl
from jax.experimental.pallas import tpu as pltpu, tpu_sc as plsc

B, S, D = 1, 2048, 768
E, F, K = 64, 384, 8
T = B * S
TM = 512
NT = T * K // TM + E
NTP = 256
NS = NT * TM
CH = 256
A = T * K
NW = 32
APW = A // NW
GC = 64
NCH = APW // GC
HD = D // 2
DW = HD

_mesh = plsc.VectorSubcoreMesh(core_axis_name="c", subcore_axis_name="s")


def _wid():
    return lax.axis_index("s") * 2 + lax.axis_index("c")


def _pack16(lo_bf16, hi_bf16):
    lo = jax.lax.bitcast_convert_type(lo_bf16, jnp.int16).astype(jnp.int32)
    hi = jax.lax.bitcast_convert_type(hi_bf16, jnp.int16).astype(jnp.int32)
    return (lo & 0xFFFF) | (hi << 16)


def _unpack16(v):
    lo_f = jax.lax.bitcast_convert_type(v << 16, jnp.float32)
    hi_f = jax.lax.bitcast_convert_type(v & jnp.int32(-65536), jnp.float32)
    return lo_f, hi_f


def _router_meta_kernel(x_ref, wg_ref, scores_ref, pos_ref, gid_ref, nt_ref,
                        xi_ref, m_ref, r_ref):
    xf = x_ref[...]
    xi_ref[...] = _pack16(xf[:, :HD].astype(jnp.bfloat16),
                          xf[:, HD:].astype(jnp.bfloat16))
    logits = jax.lax.dot_general(
        xf, wg_ref[...], (((1,), (1,)), ((), ())),
        preferred_element_type=jnp.float32)
    mx = jnp.max(logits, axis=-1, keepdims=True)
    ex = jnp.exp(logits - mx)
    probs = ex / jnp.sum(ex, axis=-1, keepdims=True)

    lane = jax.lax.broadcasted_iota(jnp.int32, (T, E), 1)
    cur = probs
    sel_any = jnp.zeros((T, E), dtype=jnp.float32)
    eidx_cols = []
    score_cols = []
    for _ in range(K):
        m = jnp.max(cur, axis=-1, keepdims=True)
        is_max = cur == m
        first = jnp.min(jnp.where(is_max, lane, E), axis=-1, keepdims=True)
        sel = lane == first
        eidx_cols.append(first)
        score_cols.append(
            jnp.sum(jnp.where(sel, probs, 0.0), axis=-1, keepdims=True))
        sel_any = jnp.where(sel, 1.0, sel_any)
        cur = jnp.where(sel, -jnp.inf, cur)
    eidx = jnp.concatenate(eidx_cols, axis=1)
    sc = jnp.concatenate(score_cols, axis=1)
    scores_ref[...] = sc / jnp.sum(sc, axis=1, keepdims=True)

    m_ref[...] = sel_any
    row = jax.lax.broadcasted_iota(jnp.int32, (CH, CH), 0)
    col = jax.lax.broadcasted_iota(jnp.int32, (CH, CH), 1)
    tril_s = jnp.where(col < row, 1.0, 0.0)

    def body(k, base):
        mc = m_ref[pl.ds(k * CH, CH), :]
        rc = jax.lax.dot_general(
            tril_s, mc, (((1,), (0,)), ((), ())),
            preferred_element_type=jnp.float32) + base
        r_ref[pl.ds(k * CH, CH), :] = rc
        return base + jnp.sum(mc, axis=0, keepdims=True)

    counts = jax.lax.fori_loop(0, T // CH, body, jnp.zeros((1, E), jnp.float32))

    tcnt = jnp.ceil(counts / TM)
    er = jax.lax.broadcasted_iota(jnp.int32, (E, E), 0)
    ec = jax.lax.broadcasted_iota(jnp.int32, (E, E), 1)
    upper_s = jnp.where(er < ec, 1.0, 0.0)
    ts_row = jax.lax.dot_general(
        tcnt, upper_s, (((1,), (0,)), ((), ())),
        preferred_element_type=jnp.float32)
    nt_ref[...] = jnp.sum(tcnt, axis=1, keepdims=True).astype(jnp.int32)

    lower_s = jnp.where(er > ec, 1.0, 0.0)
    ts_col = jax.lax.dot_general(
        lower_s, tcnt, (((1,), (1,)), ((), ())),
        preferred_element_type=jnp.float32)
    ti = jax.lax.broadcasted_iota(jnp.int32, (E, NTP), 1)
    cmp = jnp.where(ti >= ts_col.astype(jnp.int32), 1.0, 0.0)
    gid = jax.lax.dot_general(
        jnp.ones((1, E), jnp.float32), cmp, (((1,), (0,)), ((), ())),
        preferred_element_type=jnp.float32) - 1.0
    gid_ref[...] = jnp.clip(gid, 0.0, E - 1).astype(jnp.int32)

    rmat = r_ref[...]
    base_row = ts_row * TM
    for k in range(K):
        sel = lane == eidx[:, k:k + 1]
        posk = jnp.sum(jnp.where(sel, base_row + rmat, 0.0),
                       axis=1, keepdims=True)
        pos_ref[:, k:k + 1] = posk.astype(jnp.int32)


def _router_meta(x, Wg):
    return pl.pallas_call(
        _router_meta_kernel,
        out_shape=(
            jax.ShapeDtypeStruct((T, K), jnp.float32),
            jax.ShapeDtypeStruct((T, K), jnp.int32),
            jax.ShapeDtypeStruct((1, NTP), jnp.int32),
            jax.ShapeDtypeStruct((1, 1), jnp.int32),
            jax.ShapeDtypeStruct((T, DW), jnp.int32),
        ),
        scratch_shapes=[
            pltpu.VMEM((T, E), jnp.float32),
            pltpu.VMEM((T, E), jnp.float32),
        ],
    )(x, Wg)


@functools.partial(
    pl.kernel, mesh=_mesh,
    out_type=jax.ShapeDtypeStruct((NS, DW), jnp.int32),
    scratch_types=[
        pltpu.VMEM((NCH, GC), jnp.int32),
        pltpu.VMEM((APW,), jnp.int32),
        pltpu.VMEM((2, GC, DW), jnp.int32),
        pltpu.SemaphoreType.DMA,
        pltpu.SemaphoreType.DMA,
    ],
)
def _sc_dispatch(pos_hbm, x_hbm, xs_hbm, pos_v, tik_v, buf, gsem, ssem):
    base = _wid() * APW
    for c in range(NCH):
        pltpu.sync_copy(pos_hbm.at[pl.ds(base + c * GC, GC)], pos_v.at[c])
    for q in range(APW // 16):
        v = lax.broadcasted_iota(jnp.int32, (16,), 0)
        tik_v[pl.ds(q * 16, 16)] = (v + (base + q * 16)) >> 3

    def gather(c):
        return pltpu.async_copy(
            x_hbm.at[tik_v.at[pl.ds(c * GC, GC)]], buf.at[c % 2], gsem)

    def scatter(c):
        return pltpu.async_copy(
            buf.at[c % 2], xs_hbm.at[pos_v.at[c]], ssem)

    gather(0)
    for c in range(NCH):
        pltpu.make_async_copy(
            x_hbm.at[tik_v.at[pl.ds(c * GC, GC)]], buf.at[c % 2], gsem).wait()
        scatter(c)
        if c + 1 < NCH:
            if c >= 1:
                pltpu.make_async_copy(
                    buf.at[(c - 1) % 2], xs_hbm.at[pos_v.at[c - 1]],
                    ssem).wait()
            gather(c + 1)
    for c in range(NCH - 2, NCH):
        pltpu.make_async_copy(
            buf.at[c % 2], xs_hbm.at[pos_v.at[c]], ssem).wait()


def _ffn_kernel(gid_ref, nt_ref, xs_ref, w1_ref, w3_ref, w2_ref, ys_ref):
    i = pl.program_id(0)

    @pl.when(i < nt_ref[0])
    def _():
        lo_f, hi_f = _unpack16(xs_ref[...])
        xlo = lo_f.astype(jnp.bfloat16)
        xhi = hi_f.astype(jnp.bfloat16)

        def dot2(w_ref):
            return (jax.lax.dot_general(
                        xlo, w_ref[0, :HD, :].astype(jnp.bfloat16),
                        (((1,), (0,)), ((), ())),
                        preferred_element_type=jnp.float32)
                    + jax.lax.dot_general(
                        xhi, w_ref[0, HD:, :].astype(jnp.bfloat16),
                        (((1,), (0,)), ((), ())),
                        preferred_element_type=jnp.float32))

        h1 = dot2(w1_ref)
        h3 = dot2(w3_ref)
        h = (h1 * jax.lax.logistic(h1) * h3).astype(jnp.bfloat16)
        y = jax.lax.dot_general(
            h, w2_ref[0].astype(jnp.bfloat16), (((1,), (0,)), ((), ())),
            preferred_element_type=jnp.float32)
        ys_ref[...] = _pack16(y[:, :HD].astype(jnp.bfloat16),
                              y[:, HD:].astype(jnp.bfloat16))


def _grouped_ffn(xs, W1b, W3b, W2b, gid, nt):
    grid_spec = pltpu.PrefetchScalarGridSpec(
        num_scalar_prefetch=2,
        grid=(NT,),
        in_specs=[
            pl.BlockSpec((TM, DW), lambda i, g, n: (i, 0)),
            pl.BlockSpec((1, D, F), lambda i, g, n: (g[i], 0, 0)),
            pl.BlockSpec((1, D, F), lambda i, g, n: (g[i], 0, 0)),
            pl.BlockSpec((1, F, D), lambda i, g, n: (g[i], 0, 0)),
        ],
        out_specs=pl.BlockSpec((TM, DW), lambda i, g, n: (i, 0)),
    )
    return pl.pallas_call(
        _ffn_kernel,
        grid_spec=grid_spec,
        out_shape=jax.ShapeDtypeStruct((NS, DW), jnp.int32),
        compiler_params=pltpu.CompilerParams(
            dimension_semantics=("arbitrary",),
        ),
    )(gid, nt, xs, W1b, W3b, W2b)


@functools.partial(
    pl.kernel, mesh=_mesh,
    out_type=jax.ShapeDtypeStruct((A, DW), jnp.int32),
    scratch_types=[
        pltpu.VMEM((APW,), jnp.int32),
        pltpu.VMEM((2, GC, DW), jnp.int32),
        pltpu.SemaphoreType.DMA,
        pltpu.SemaphoreType.DMA,
    ],
)
def _sc_gather_back(pos_hbm, ys_hbm, yg_hbm, pos_v, buf, gsem, wsem):
    base = _wid() * APW
    pltpu.sync_copy(pos_hbm.at[pl.ds(base, APW)], pos_v)

    def gather(c):
        return pltpu.async_copy(
            ys_hbm.at[pos_v.at[pl.ds(c * GC, GC)]], buf.at[c % 2], gsem)

    def wout(c):
        return pltpu.async_copy(
            buf.at[c % 2], yg_hbm.at[pl.ds(base + c * GC, GC)], wsem)

    gather(0)
    for c in range(NCH):
        pltpu.make_async_copy(
            ys_hbm.at[pos_v.at[pl.ds(c * GC, GC)]], buf.at[c % 2], gsem).wait()
        wout(c)
        if c + 1 < NCH:
            if c >= 1:
                pltpu.make_async_copy(
                    buf.at[(c - 1) % 2],
                    yg_hbm.at[pl.ds(base + (c - 1) * GC, GC)], wsem).wait()
            gather(c + 1)
    for c in range(NCH - 2, NCH):
        pltpu.make_async_copy(
            buf.at[c % 2], yg_hbm.at[pl.ds(base + c * GC, GC)], wsem).wait()


def _combine_kernel(yg_ref, sc_ref, out_ref):
    lo_f, hi_f = _unpack16(yg_ref[:, 0:DW])
    acc_lo = sc_ref[:, 0:1] * lo_f
    acc_hi = sc_ref[:, 0:1] * hi_f
    for k in range(1, K):
        lo_f, hi_f = _unpack16(yg_ref[:, k * DW:(k + 1) * DW])
        acc_lo = acc_lo + sc_ref[:, k:k + 1] * lo_f
        acc_hi = acc_hi + sc_ref[:, k:k + 1] * hi_f
    out_ref[:, :HD] = acc_lo
    out_ref[:, HD:] = acc_hi


def _combine(yg, scores):
    CB = 256
    return pl.pallas_call(
        _combine_kernel,
        grid=(T // CB,),
        in_specs=[
            pl.BlockSpec((CB, K * DW), lambda i: (i, 0)),
            pl.BlockSpec((CB, K), lambda i: (i, 0)),
        ],
        out_specs=pl.BlockSpec((CB, D), lambda i: (i, 0)),
        out_shape=jax.ShapeDtypeStruct((T, D), jnp.float32),
    )(yg, scores)


def kernel(hidden_states, Wg, W1, W3, W2):
    old_shape = hidden_states.shape
    x = hidden_states.reshape(-1, old_shape[-1])

    scores, pos, gid, nt, xi = _router_meta(x, Wg)
    gid = gid.reshape(NTP)
    nt = nt.reshape(1)
    pos_flat = pos.reshape(A)

    xs = _sc_dispatch(pos_flat, xi)
    ys = _grouped_ffn(xs, W1, W3, W2, gid, nt)
    yg = _sc_gather_back(pos_flat, ys)
    out = _combine(yg.reshape(T, K * DW), scores)
    return out.reshape(old_shape)


# speedup vs baseline: 1.4763x; 1.0239x over previous
import functools

import jax
import jax.numpy as jnp
from jax import lax
from jax.experimental import pallas as pl
from jax.experimental.pallas import tpu as pltpu, tpu_sc as plsc

B, S, D = 1, 2048, 768
E, F, K = 64, 384, 8
T = B * S
TM = 512
NT = T * K // TM + E
NTP = 256
NS = NT * TM
CH = 256
A = T * K
NW = 32
APW = A // NW
GC = 128
NCH = APW // GC
HD = D // 2
DW = HD

_mesh = plsc.VectorSubcoreMesh(core_axis_name="c", subcore_axis_name="s")


def _wid():
    return lax.axis_index("s") * 2 + lax.axis_index("c")


def _pack16(lo_bf16, hi_bf16):
    lo = jax.lax.bitcast_convert_type(lo_bf16, jnp.int16).astype(jnp.int32)
    hi = jax.lax.bitcast_convert_type(hi_bf16, jnp.int16).astype(jnp.int32)
    return (lo & 0xFFFF) | (hi << 16)


def _unpack16(v):
    lo_f = jax.lax.bitcast_convert_type(v << 16, jnp.float32)
    hi_f = jax.lax.bitcast_convert_type(v & jnp.int32(-65536), jnp.float32)
    return lo_f, hi_f


def _router_meta_kernel(x_ref, wg_ref, scores_ref, pos_ref, gid_ref, nt_ref,
                        xi_ref, m_ref, r_ref):
    xf = x_ref[...]
    xi_ref[...] = _pack16(xf[:, :HD].astype(jnp.bfloat16),
                          xf[:, HD:].astype(jnp.bfloat16))
    logits = jax.lax.dot_general(
        xf, wg_ref[...], (((1,), (1,)), ((), ())),
        preferred_element_type=jnp.float32)
    mx = jnp.max(logits, axis=-1, keepdims=True)
    ex = jnp.exp(logits - mx)
    probs = ex / jnp.sum(ex, axis=-1, keepdims=True)

    lane = jax.lax.broadcasted_iota(jnp.int32, (T, E), 1)
    cur = probs
    sel_any = jnp.zeros((T, E), dtype=jnp.float32)
    eidx_cols = []
    score_cols = []
    for _ in range(K):
        m = jnp.max(cur, axis=-1, keepdims=True)
        is_max = cur == m
        first = jnp.min(jnp.where(is_max, lane, E), axis=-1, keepdims=True)
        sel = lane == first
        eidx_cols.append(first)
        score_cols.append(
            jnp.sum(jnp.where(sel, probs, 0.0), axis=-1, keepdims=True))
        sel_any = jnp.where(sel, 1.0, sel_any)
        cur = jnp.where(sel, -jnp.inf, cur)
    eidx = jnp.concatenate(eidx_cols, axis=1)
    sc = jnp.concatenate(score_cols, axis=1)
    scores_ref[...] = sc / jnp.sum(sc, axis=1, keepdims=True)

    m_ref[...] = sel_any
    row = jax.lax.broadcasted_iota(jnp.int32, (CH, CH), 0)
    col = jax.lax.broadcasted_iota(jnp.int32, (CH, CH), 1)
    tril_s = jnp.where(col < row, 1.0, 0.0)

    def body(k, base):
        mc = m_ref[pl.ds(k * CH, CH), :]
        rc = jax.lax.dot_general(
            tril_s, mc, (((1,), (0,)), ((), ())),
            preferred_element_type=jnp.float32) + base
        r_ref[pl.ds(k * CH, CH), :] = rc
        return base + jnp.sum(mc, axis=0, keepdims=True)

    counts = jax.lax.fori_loop(0, T // CH, body, jnp.zeros((1, E), jnp.float32))

    tcnt = jnp.ceil(counts / TM)
    er = jax.lax.broadcasted_iota(jnp.int32, (E, E), 0)
    ec = jax.lax.broadcasted_iota(jnp.int32, (E, E), 1)
    upper_s = jnp.where(er < ec, 1.0, 0.0)
    ts_row = jax.lax.dot_general(
        tcnt, upper_s, (((1,), (0,)), ((), ())),
        preferred_element_type=jnp.float32)
    nt_ref[...] = jnp.sum(tcnt, axis=1, keepdims=True).astype(jnp.int32)

    lower_s = jnp.where(er > ec, 1.0, 0.0)
    ts_col = jax.lax.dot_general(
        lower_s, tcnt, (((1,), (1,)), ((), ())),
        preferred_element_type=jnp.float32)
    ti = jax.lax.broadcasted_iota(jnp.int32, (E, NTP), 1)
    cmp = jnp.where(ti >= ts_col.astype(jnp.int32), 1.0, 0.0)
    gid = jax.lax.dot_general(
        jnp.ones((1, E), jnp.float32), cmp, (((1,), (0,)), ((), ())),
        preferred_element_type=jnp.float32) - 1.0
    gid_ref[...] = jnp.clip(gid, 0.0, E - 1).astype(jnp.int32)

    rmat = r_ref[...]
    base_row = ts_row * TM
    for k in range(K):
        sel = lane == eidx[:, k:k + 1]
        posk = jnp.sum(jnp.where(sel, base_row + rmat, 0.0),
                       axis=1, keepdims=True)
        pos_ref[:, k:k + 1] = posk.astype(jnp.int32)


def _router_meta(x, Wg):
    return pl.pallas_call(
        _router_meta_kernel,
        out_shape=(
            jax.ShapeDtypeStruct((T, K), jnp.float32),
            jax.ShapeDtypeStruct((T, K), jnp.int32),
            jax.ShapeDtypeStruct((1, NTP), jnp.int32),
            jax.ShapeDtypeStruct((1, 1), jnp.int32),
            jax.ShapeDtypeStruct((T, DW), jnp.int32),
        ),
        scratch_shapes=[
            pltpu.VMEM((T, E), jnp.float32),
            pltpu.VMEM((T, E), jnp.float32),
        ],
    )(x, Wg)


@functools.partial(
    pl.kernel, mesh=_mesh,
    out_type=jax.ShapeDtypeStruct((NS, DW), jnp.int32),
    scratch_types=[
        pltpu.VMEM((NCH, GC), jnp.int32),
        pltpu.VMEM((APW,), jnp.int32),
        pltpu.VMEM((2, GC, DW), jnp.int32),
        pltpu.SemaphoreType.DMA,
        pltpu.SemaphoreType.DMA,
    ],
)
def _sc_dispatch(pos_hbm, x_hbm, xs_hbm, pos_v, tik_v, buf, gsem, ssem):
    base = _wid() * APW
    for c in range(NCH):
        pltpu.sync_copy(pos_hbm.at[pl.ds(base + c * GC, GC)], pos_v.at[c])
    for q in range(APW // 16):
        v = lax.broadcasted_iota(jnp.int32, (16,), 0)
        tik_v[pl.ds(q * 16, 16)] = (v + (base + q * 16)) >> 3

    def gather(c):
        return pltpu.async_copy(
            x_hbm.at[tik_v.at[pl.ds(c * GC, GC)]], buf.at[c % 2], gsem)

    def scatter(c):
        return pltpu.async_copy(
            buf.at[c % 2], xs_hbm.at[pos_v.at[c]], ssem)

    gather(0)
    for c in range(NCH):
        pltpu.make_async_copy(
            x_hbm.at[tik_v.at[pl.ds(c * GC, GC)]], buf.at[c % 2], gsem).wait()
        scatter(c)
        if c + 1 < NCH:
            if c >= 1:
                pltpu.make_async_copy(
                    buf.at[(c - 1) % 2], xs_hbm.at[pos_v.at[c - 1]],
                    ssem).wait()
            gather(c + 1)
    for c in range(NCH - 2, NCH):
        pltpu.make_async_copy(
            buf.at[c % 2], xs_hbm.at[pos_v.at[c]], ssem).wait()


def _ffn_kernel(gid_ref, nt_ref, xs_ref, w1_ref, w3_ref, w2_ref, ys_ref):
    i = pl.program_id(0)

    @pl.when(i < nt_ref[0])
    def _():
        lo_f, hi_f = _unpack16(xs_ref[...])
        xlo = lo_f.astype(jnp.bfloat16)
        xhi = hi_f.astype(jnp.bfloat16)

        def dot2(w_ref):
            return (jax.lax.dot_general(
                        xlo, w_ref[0, :HD, :].astype(jnp.bfloat16),
                        (((1,), (0,)), ((), ())),
                        preferred_element_type=jnp.float32)
                    + jax.lax.dot_general(
                        xhi, w_ref[0, HD:, :].astype(jnp.bfloat16),
                        (((1,), (0,)), ((), ())),
                        preferred_element_type=jnp.float32))

        h1 = dot2(w1_ref)
        h3 = dot2(w3_ref)
        h = (h1 * jax.lax.logistic(h1) * h3).astype(jnp.bfloat16)
        y = jax.lax.dot_general(
            h, w2_ref[0].astype(jnp.bfloat16), (((1,), (0,)), ((), ())),
            preferred_element_type=jnp.float32)
        ys_ref[...] = _pack16(y[:, :HD].astype(jnp.bfloat16),
                              y[:, HD:].astype(jnp.bfloat16))


def _grouped_ffn(xs, W1b, W3b, W2b, gid, nt):
    grid_spec = pltpu.PrefetchScalarGridSpec(
        num_scalar_prefetch=2,
        grid=(NT,),
        in_specs=[
            pl.BlockSpec((TM, DW), lambda i, g, n: (i, 0)),
            pl.BlockSpec((1, D, F), lambda i, g, n: (g[i], 0, 0)),
            pl.BlockSpec((1, D, F), lambda i, g, n: (g[i], 0, 0)),
            pl.BlockSpec((1, F, D), lambda i, g, n: (g[i], 0, 0)),
        ],
        out_specs=pl.BlockSpec((TM, DW), lambda i, g, n: (i, 0)),
    )
    return pl.pallas_call(
        _ffn_kernel,
        grid_spec=grid_spec,
        out_shape=jax.ShapeDtypeStruct((NS, DW), jnp.int32),
        compiler_params=pltpu.CompilerParams(
            dimension_semantics=("arbitrary",),
        ),
    )(gid, nt, xs, W1b, W3b, W2b)


@functools.partial(
    pl.kernel, mesh=_mesh,
    out_type=jax.ShapeDtypeStruct((A, DW), jnp.int32),
    scratch_types=[
        pltpu.VMEM((APW,), jnp.int32),
        pltpu.VMEM((2, GC, DW), jnp.int32),
        pltpu.SemaphoreType.DMA,
        pltpu.SemaphoreType.DMA,
    ],
)
def _sc_gather_back(pos_hbm, ys_hbm, yg_hbm, pos_v, buf, gsem, wsem):
    base = _wid() * APW
    pltpu.sync_copy(pos_hbm.at[pl.ds(base, APW)], pos_v)

    def gather(c):
        return pltpu.async_copy(
            ys_hbm.at[pos_v.at[pl.ds(c * GC, GC)]], buf.at[c % 2], gsem)

    def wout(c):
        return pltpu.async_copy(
            buf.at[c % 2], yg_hbm.at[pl.ds(base + c * GC, GC)], wsem)

    gather(0)
    for c in range(NCH):
        pltpu.make_async_copy(
            ys_hbm.at[pos_v.at[pl.ds(c * GC, GC)]], buf.at[c % 2], gsem).wait()
        wout(c)
        if c + 1 < NCH:
            if c >= 1:
                pltpu.make_async_copy(
                    buf.at[(c - 1) % 2],
                    yg_hbm.at[pl.ds(base + (c - 1) * GC, GC)], wsem).wait()
            gather(c + 1)
    for c in range(NCH - 2, NCH):
        pltpu.make_async_copy(
            buf.at[c % 2], yg_hbm.at[pl.ds(base + c * GC, GC)], wsem).wait()


def _combine_kernel(yg_ref, sc_ref, out_ref):
    lo_f, hi_f = _unpack16(yg_ref[:, 0:DW])
    acc_lo = sc_ref[:, 0:1] * lo_f
    acc_hi = sc_ref[:, 0:1] * hi_f
    for k in range(1, K):
        lo_f, hi_f = _unpack16(yg_ref[:, k * DW:(k + 1) * DW])
        acc_lo = acc_lo + sc_ref[:, k:k + 1] * lo_f
        acc_hi = acc_hi + sc_ref[:, k:k + 1] * hi_f
    out_ref[:, :HD] = acc_lo
    out_ref[:, HD:] = acc_hi


def _combine(yg, scores):
    CB = 256
    return pl.pallas_call(
        _combine_kernel,
        grid=(T // CB,),
        in_specs=[
            pl.BlockSpec((CB, K * DW), lambda i: (i, 0)),
            pl.BlockSpec((CB, K), lambda i: (i, 0)),
        ],
        out_specs=pl.BlockSpec((CB, D), lambda i: (i, 0)),
        out_shape=jax.ShapeDtypeStruct((T, D), jnp.float32),
    )(yg, scores)


def kernel(hidden_states, Wg, W1, W3, W2):
    old_shape = hidden_states.shape
    x = hidden_states.reshape(-1, old_shape[-1])

    scores, pos, gid, nt, xi = _router_meta(x, Wg)
    gid = gid.reshape(NTP)
    nt = nt.reshape(1)
    pos_flat = pos.reshape(A)

    xs = _sc_dispatch(pos_flat, xi)
    ys = _grouped_ffn(xs, W1, W3, W2, gid, nt)
    yg = _sc_gather_back(pos_flat, ys)
    out = _combine(yg.reshape(T, K * DW), scores)
    return out.reshape(old_shape)
